# Initial kernel scaffold; baseline (speedup 1.0000x reference)
import jax
import jax.numpy as jnp
from jax.experimental import pallas as pl


def kernel(input_ids, attention_mask):
    raise NotImplementedError("write your pallas kernel here")


# speedup vs baseline: 9.3865x; 9.3865x over previous
import jax
import jax.numpy as jnp
from jax import lax
from jax.experimental import pallas as pl

_VD = 128
_BB = 16


def _onehot_block(ids_ref, out_ref):
    ids = ids_ref[...]
    bb, s = ids.shape
    pos = lax.broadcasted_iota(jnp.int32, (bb, s), 1)
    idx1 = jnp.mod(ids, _VD)
    idx2 = jnp.mod(ids * 37 + pos * 11, _VD)
    lane = lax.broadcasted_iota(jnp.int32, (bb, s, _VD), 2)
    out = jnp.where(lane == idx1[:, :, None], jnp.float32(1.0), jnp.float32(0.0))
    out = out + jnp.where(lane == idx2[:, :, None], jnp.float32(0.5), jnp.float32(0.0))
    out_ref[...] = out


def kernel(input_ids, attention_mask):
    del attention_mask
    B, S = input_ids.shape
    grid = (B // _BB,)
    return pl.pallas_call(
        _onehot_block,
        grid=grid,
        in_specs=[pl.BlockSpec((_BB, S), lambda i: (i, 0))],
        out_specs=pl.BlockSpec((_BB, S, _VD), lambda i: (i, 0, 0)),
        out_shape=jax.ShapeDtypeStruct((B, S, _VD), jnp.float32),
    )(input_ids.astype(jnp.int32))


# speedup vs baseline: 11.0044x; 1.1724x over previous
import jax
import jax.numpy as jnp
from jax import lax
from jax.experimental import pallas as pl

_VD = 128
_BB = 32


def _onehot_block(ids_ref, out_ref):
    ids = ids_ref[...]
    bb, s = ids.shape
    pos = lax.broadcasted_iota(jnp.int32, (bb, s), 1)
    idx1 = jnp.mod(ids, _VD)
    idx2 = jnp.mod(ids * 37 + pos * 11, _VD)
    lane = lax.broadcasted_iota(jnp.int32, (bb, s, _VD), 2)
    out = jnp.where(lane == idx1[:, :, None], jnp.float32(1.0), jnp.float32(0.0))
    out = out + jnp.where(lane == idx2[:, :, None], jnp.float32(0.5), jnp.float32(0.0))
    out_ref[...] = out


def kernel(input_ids, attention_mask):
    del attention_mask
    B, S = input_ids.shape
    grid = (B // _BB,)
    return pl.pallas_call(
        _onehot_block,
        grid=grid,
        in_specs=[pl.BlockSpec((_BB, S), lambda i: (i, 0))],
        out_specs=pl.BlockSpec((_BB, S, _VD), lambda i: (i, 0, 0)),
        out_shape=jax.ShapeDtypeStruct((B, S, _VD), jnp.float32),
    )(input_ids.astype(jnp.int32))


# speedup vs baseline: 11.0947x; 1.0082x over previous
import jax
import jax.numpy as jnp
from jax import lax
from jax.experimental import pallas as pl

_VD = 128
_BB = 64


def _onehot_block(ids_ref, out_ref):
    ids = ids_ref[...]
    bb, s = ids.shape
    pos = lax.broadcasted_iota(jnp.int32, (bb, s), 1)
    idx1 = jnp.mod(ids, _VD)
    idx2 = jnp.mod(ids * 37 + pos * 11, _VD)
    lane = lax.broadcasted_iota(jnp.int32, (bb, s, _VD), 2)
    out = jnp.where(lane == idx1[:, :, None], jnp.float32(1.0), jnp.float32(0.0))
    out = out + jnp.where(lane == idx2[:, :, None], jnp.float32(0.5), jnp.float32(0.0))
    out_ref[...] = out


def kernel(input_ids, attention_mask):
    del attention_mask
    B, S = input_ids.shape
    grid = (B // _BB,)
    return pl.pallas_call(
        _onehot_block,
        grid=grid,
        in_specs=[pl.BlockSpec((_BB, S), lambda i: (i, 0))],
        out_specs=pl.BlockSpec((_BB, S, _VD), lambda i: (i, 0, 0)),
        out_shape=jax.ShapeDtypeStruct((B, S, _VD), jnp.float32),
    )(input_ids.astype(jnp.int32))


# speedup vs baseline: 16.5914x; 1.4954x over previous
import jax
import jax.numpy as jnp
from jax import lax
from jax.experimental import pallas as pl

_VD = 128
_BB = 64


def _onehot_block(ids_ref, out_ref):
    ids = ids_ref[...]
    bb, s = ids.shape
    pos = lax.broadcasted_iota(jnp.int32, (bb, s), 1)
    idx1 = jnp.mod(ids, _VD)
    idx2 = jnp.mod(ids * 37 + pos * 11, _VD)
    packed = jnp.bitwise_or(idx1, jnp.left_shift(idx2, 8))
    pk = jnp.broadcast_to(packed[:, :, None], (bb, s, _VD))
    lane = lax.broadcasted_iota(jnp.int32, (bb, s, _VD), 2)
    eq1 = jnp.bitwise_and(pk, 0xFF) == lane
    eq2 = jnp.right_shift(pk, 8) == lane
    out = jnp.where(eq1, jnp.float32(1.0), jnp.float32(0.0))
    out = out + jnp.where(eq2, jnp.float32(0.5), jnp.float32(0.0))
    out_ref[...] = out


def kernel(input_ids, attention_mask):
    del attention_mask
    B, S = input_ids.shape
    grid = (B // _BB,)
    return pl.pallas_call(
        _onehot_block,
        grid=grid,
        in_specs=[pl.BlockSpec((_BB, S), lambda i: (i, 0))],
        out_specs=pl.BlockSpec((_BB, S, _VD), lambda i: (i, 0, 0)),
        out_shape=jax.ShapeDtypeStruct((B, S, _VD), jnp.float32),
    )(input_ids.astype(jnp.int32))


# speedup vs baseline: 16.9052x; 1.0189x over previous
import jax
import jax.numpy as jnp
from jax import lax
from jax.experimental import pallas as pl

_VD = 128
_BB = 128


def _onehot_block(ids_ref, out_ref):
    ids = ids_ref[...]
    bb, s = ids.shape
    pos = lax.broadcasted_iota(jnp.int32, (bb, s), 1)
    idx1 = jnp.mod(ids, _VD)
    idx2 = jnp.mod(ids * 37 + pos * 11, _VD)
    packed = jnp.bitwise_or(idx1, jnp.left_shift(idx2, 8))
    pk = jnp.broadcast_to(packed[:, :, None], (bb, s, _VD))
    lane = lax.broadcasted_iota(jnp.int32, (bb, s, _VD), 2)
    eq1 = jnp.bitwise_and(pk, 0xFF) == lane
    eq2 = jnp.right_shift(pk, 8) == lane
    out = jnp.where(eq1, jnp.float32(1.0), jnp.float32(0.0))
    out = out + jnp.where(eq2, jnp.float32(0.5), jnp.float32(0.0))
    out_ref[...] = out


def kernel(input_ids, attention_mask):
    del attention_mask
    B, S = input_ids.shape
    grid = (B // _BB,)
    return pl.pallas_call(
        _onehot_block,
        grid=grid,
        in_specs=[pl.BlockSpec((_BB, S), lambda i: (i, 0))],
        out_specs=pl.BlockSpec((_BB, S, _VD), lambda i: (i, 0, 0)),
        out_shape=jax.ShapeDtypeStruct((B, S, _VD), jnp.float32),
    )(input_ids.astype(jnp.int32))
